# baseline (speedup 1.0000x reference)
import functools

import jax
import jax.numpy as jnp
from jax import lax
from jax.experimental import pallas as pl
from jax.experimental.pallas import tpu as pltpu
from jax.experimental.pallas import tpu_sc as plsc

_BATCH = 16
_H = 64
_W = 64


def kernel(batch, length_h, length_w, embeddings_tablex, embeddings_tabley):
    n, feat = embeddings_tablex.shape
    cat = jnp.concatenate(
        [embeddings_tablex, jnp.flip(embeddings_tablex, axis=0)], axis=1
    ).reshape(-1)

    info = plsc.get_sparse_core_info()
    nw = info.num_cores * info.num_subcores
    jobs_per_w = (_BATCH * _H) // nw
    i_span = _H // (nw // _BATCH)

    mesh = plsc.VectorSubcoreMesh(core_axis_name="c", subcore_axis_name="s")

    slab = _W * 2 * feat

    @functools.partial(
        pl.kernel,
        out_type=jax.ShapeDtypeStruct((_BATCH * _H * _W * 2 * feat,), jnp.float32),
        mesh=mesh,
        scratch_types=[
            pltpu.VMEM((n * 2 * feat,), jnp.float32),
            pltpu.SemaphoreType.DMA,
        ],
    )
    def relpos(cat_hbm, out_hbm, cat_v, sem):
        wid = lax.axis_index("s") * info.num_cores + lax.axis_index("c")
        pltpu.sync_copy(cat_hbm, cat_v)
        b = wid // (nw // _BATCH)
        i0 = (wid % (nw // _BATCH)) * i_span
        copies = []
        for t in range(jobs_per_w):
            i = i0 + t
            row = b * (_H * _W) + i * _W
            copies.append(
                pltpu.async_copy(
                    cat_v.at[pl.ds(((_H - i0) - t) * 2 * feat, slab)],
                    out_hbm.at[pl.ds(row * 2 * feat, slab)],
                    sem,
                )
            )
        for c in copies:
            c.wait()

    out = relpos(cat)
    return out.reshape(_BATCH, _H * _W, 2 * feat)
